# Initial kernel scaffold; baseline (speedup 1.0000x reference)
import functools

import jax
import jax.numpy as jnp
from jax import lax
from jax.experimental import pallas as pl
from jax.experimental.pallas import tpu as pltpu
from jax.experimental.pallas import tpu_sc as plsc

N = 10000
E = 320000
FH = 128

NC = 2
NS = 16
LANES = 16
NW = NC * NS
EPT = E // NW
CHUNK = 400
NCHUNK = EPT // CHUNK
ROWS_PT = N // NS

_mesh = plsc.VectorSubcoreMesh(core_axis_name="c", subcore_axis_name="s")


def _deg_body(src_hbm, dst_hbm, out_hbm, src_v, dst_v, outdeg_v, indeg_v, sem):
    cid = lax.axis_index("c")
    sid = lax.axis_index("s")
    wid = cid * NS + sid
    pltpu.async_copy(src_hbm.at[wid], src_v, sem).wait()
    pltpu.async_copy(dst_hbm.at[wid], dst_v, sem).wait()

    zero16 = jnp.zeros((LANES,), jnp.float32)
    ones16 = jnp.full((LANES,), 1.0, jnp.float32)

    @pl.loop(0, N, step=LANES)
    def _(j):
        outdeg_v[pl.ds(j, LANES)] = zero16
        indeg_v[pl.ds(j, LANES)] = zero16

    @pl.loop(0, NCHUNK)
    def _(c):
        @pl.loop(0, CHUNK, step=LANES)
        def _(i):
            s16 = src_v[c, pl.ds(i, LANES)]
            d16 = dst_v[c, pl.ds(i, LANES)]
            plsc.addupdate_scatter(outdeg_v, [s16], ones16)
            plsc.addupdate_scatter(indeg_v, [d16], ones16)

    pltpu.async_copy(outdeg_v, out_hbm.at[wid, 0], sem).wait()
    pltpu.async_copy(indeg_v, out_hbm.at[wid, 1], sem).wait()


_deg_call = pl.kernel(
    _deg_body,
    out_type=jax.ShapeDtypeStruct((NW, 2, N), jnp.float32),
    mesh=_mesh,
    scratch_types=[
        pltpu.VMEM((NCHUNK, CHUNK), jnp.int32),
        pltpu.VMEM((NCHUNK, CHUNK), jnp.int32),
        pltpu.VMEM((N,), jnp.float32),
        pltpu.VMEM((N,), jnp.float32),
        pltpu.SemaphoreType.DMA,
    ],
)


def _mm1_body(deg_ref, x_ref, w1_ref, h1s_ref, norms_ref):
    deg = jnp.sum(deg_ref[...], axis=0)
    norms = lax.rsqrt(jnp.maximum(deg, 1.0))
    norms_ref[...] = norms
    xs = x_ref[...] * norms[0][:, None]
    h1s_ref[...] = jnp.dot(xs, w1_ref[...], preferred_element_type=jnp.float32)


def _mm1_call(deg, x, w1):
    return pl.pallas_call(
        _mm1_body,
        out_shape=[
            jax.ShapeDtypeStruct((N, FH), jnp.float32),
            jax.ShapeDtypeStruct((2, N), jnp.float32),
        ],
    )(deg, x, w1)


def _agg_body(src_hbm, dst_hbm, h1s_hbm, norms_hbm, zeros_hbm, agg_hbm,
              coef_hbm, src_v, dst_v, rows_v, ndst_v, coef_v, shared_agg, sem):
    cid = lax.axis_index("c")
    sid = lax.axis_index("s")
    wid = cid * NS + sid

    pltpu.async_copy(src_hbm.at[wid], src_v, sem).wait()
    pltpu.async_copy(dst_hbm.at[wid], dst_v, sem).wait()
    pltpu.async_copy(norms_hbm.at[1], ndst_v, sem).wait()

    zero16 = jnp.zeros((LANES,), jnp.float32)

    @pl.loop(0, N, step=LANES)
    def _(j):
        coef_v[pl.ds(j, LANES)] = zero16

    pltpu.async_copy(
        zeros_hbm.at[pl.ds(sid * ROWS_PT, ROWS_PT)],
        shared_agg.at[pl.ds(sid * ROWS_PT, ROWS_PT)],
        sem,
    ).wait()
    plsc.subcore_barrier()

    @pl.loop(0, NCHUNK)
    def _(c):
        pltpu.async_copy(h1s_hbm.at[src_v.at[c]], rows_v, sem).wait()
        pltpu.sync_copy(rows_v, shared_agg.at[dst_v.at[c]], add=True)

        @pl.loop(0, CHUNK, step=LANES)
        def _(i):
            d16 = dst_v[c, pl.ds(i, LANES)]
            s16 = src_v[c, pl.ds(i, LANES)]
            vals = plsc.load_gather(ndst_v, [d16])
            plsc.addupdate_scatter(coef_v, [s16], vals)

    plsc.subcore_barrier()
    pltpu.async_copy(
        shared_agg.at[pl.ds(sid * ROWS_PT, ROWS_PT)],
        agg_hbm.at[cid, pl.ds(sid * ROWS_PT, ROWS_PT)],
        sem,
    ).wait()
    pltpu.async_copy(coef_v, coef_hbm.at[wid], sem).wait()


_agg_call = pl.kernel(
    _agg_body,
    out_type=[
        jax.ShapeDtypeStruct((NC, N, FH), jnp.float32),
        jax.ShapeDtypeStruct((NW, N), jnp.float32),
    ],
    mesh=_mesh,
    scratch_types=[
        pltpu.VMEM((NCHUNK, CHUNK), jnp.int32),
        pltpu.VMEM((NCHUNK, CHUNK), jnp.int32),
        pltpu.VMEM((CHUNK, FH), jnp.float32),
        pltpu.VMEM((N,), jnp.float32),
        pltpu.VMEM((N,), jnp.float32),
        pltpu.VMEM_SHARED((N, FH), jnp.float32),
        pltpu.SemaphoreType.DMA,
    ],
)


def _final_body(agg_ref, coef_ref, norms_ref, b1_ref, w2_ref, b2_ref, out_ref):
    agg = agg_ref[0] + agg_ref[1]
    h = jnp.maximum(agg * norms_ref[1][:, None] + b1_ref[...], 0.0)
    coef = jnp.sum(coef_ref[...], axis=0)
    w = coef * norms_ref[0]
    ws = jnp.sum(h * w[:, None], axis=0, keepdims=True)
    total = jnp.sum(ws * w2_ref[...])
    out_ref[...] = total * (1.0 / N) + b2_ref[...]


def _final_call(aggp, coefp, norms, b1r, w2r, b2r):
    return pl.pallas_call(
        _final_body,
        out_shape=jax.ShapeDtypeStruct((1, 1), jnp.float32),
    )(aggp, coefp, norms, b1r, w2r, b2r)


@jax.jit
def _gcn(x, edge_index, W1, b1, W2, b2):
    ei = edge_index.astype(jnp.int32).reshape(2, NW, NCHUNK, CHUNK)
    src4 = ei[0]
    dst4 = ei[1]
    deg = _deg_call(src4, dst4)
    h1s, norms = _mm1_call(deg, x, W1)
    zeros = jnp.zeros((N, FH), jnp.float32)
    aggp, coefp = _agg_call(src4, dst4, h1s, norms, zeros)
    b1r = b1.reshape(1, FH)
    w2r = W2.reshape(1, FH)
    b2r = b2.reshape(1, 1)
    return _final_call(aggp, coefp, norms, b1r, w2r, b2r)


def kernel(x, edge_index, W1, b1, W2, b2):
    return _gcn(x, edge_index, W1, b1, W2, b2)


# speedup vs baseline: 20.8205x; 20.8205x over previous
import dataclasses
import functools

import jax
import jax.numpy as jnp
from jax import lax
from jax.experimental import pallas as pl
from jax.experimental.pallas import tpu as pltpu
from jax.experimental.pallas import tpu_sc as plsc

N = 10000
E = 320000
FH = 128
FHALF = FH // 2

NC = 2
NS = 16
LANES = 16
NW = NC * NS
CHUNK = 400
NCHUNK1 = E // NW // CHUNK
NCHUNK2 = E // NS // CHUNK
ROWS_PT = 624
ROWS_TAIL = N - ROWS_PT * NS

_mesh = plsc.VectorSubcoreMesh(core_axis_name="c", subcore_axis_name="s")

_sc_params = pltpu.CompilerParams()
if "needs_layout_passes" in pltpu.CompilerParams.__dataclass_fields__:
    _sc_params = dataclasses.replace(_sc_params, needs_layout_passes=False)
_sc_params_untiled = _sc_params
if "use_tc_tiling_on_sc" in pltpu.CompilerParams.__dataclass_fields__:
    _sc_params_untiled = dataclasses.replace(
        _sc_params, use_tc_tiling_on_sc=False)


def _deg_body(src_hbm, dst_hbm, outdeg_hbm, indeg_hbm,
              src_v, dst_v, outdeg_v, indeg_v, sem):
    cid = lax.axis_index("c")
    sid = lax.axis_index("s")
    wid = cid * NS + sid
    pltpu.async_copy(src_hbm.at[wid], src_v, sem).wait()
    pltpu.async_copy(dst_hbm.at[wid], dst_v, sem).wait()

    zero16 = jnp.zeros((LANES,), jnp.float32)
    ones16 = jnp.full((LANES,), 1.0, jnp.float32)

    @pl.loop(0, N, step=LANES)
    def _(j):
        outdeg_v[pl.ds(j, LANES)] = zero16
        indeg_v[pl.ds(j, LANES)] = zero16

    @pl.loop(0, NCHUNK1)
    def _(c):
        @pl.loop(0, CHUNK, step=LANES)
        def _(i):
            s16 = src_v[c, 0, pl.ds(i, LANES)]
            d16 = dst_v[c, 0, pl.ds(i, LANES)]
            plsc.addupdate_scatter(outdeg_v, [s16], ones16)
            plsc.addupdate_scatter(indeg_v, [d16], ones16)

    pltpu.async_copy(outdeg_v, outdeg_hbm.at[wid, 0], sem).wait()
    pltpu.async_copy(indeg_v, indeg_hbm.at[wid, 0], sem).wait()


_deg_call = pl.kernel(
    _deg_body,
    out_type=[
        jax.ShapeDtypeStruct((NW, 1, N), jnp.float32),
        jax.ShapeDtypeStruct((NW, 1, N), jnp.float32),
    ],
    mesh=_mesh,
    scratch_types=[
        pltpu.VMEM((NCHUNK1, 1, CHUNK), jnp.int32),
        pltpu.VMEM((NCHUNK1, 1, CHUNK), jnp.int32),
        pltpu.VMEM((N,), jnp.float32),
        pltpu.VMEM((N,), jnp.float32),
        pltpu.SemaphoreType.DMA,
    ],
    compiler_params=_sc_params,
)


def _mm1_body(outdeg_ref, indeg_ref, x_ref, w1_ref, h1lo_ref, h1hi_ref,
              nsrc_ref, ndst_ref):
    dsrc = jnp.sum(outdeg_ref[...], axis=(0, 1))
    ddst = jnp.sum(indeg_ref[...], axis=(0, 1))
    nsrc = lax.rsqrt(jnp.maximum(dsrc, 1.0))
    ndst = lax.rsqrt(jnp.maximum(ddst, 1.0))
    nsrc_ref[...] = nsrc[None, :]
    ndst_ref[...] = ndst[None, :]
    xs = x_ref[...] * nsrc[:, None]
    h1s = jnp.dot(xs, w1_ref[...], preferred_element_type=jnp.float32)
    h1lo_ref[...] = h1s[:, :FHALF]
    h1hi_ref[...] = h1s[:, FHALF:]


def _mm1_call(outdeg_p, indeg_p, x, w1):
    return pl.pallas_call(
        _mm1_body,
        out_shape=[
            jax.ShapeDtypeStruct((N, FHALF), jnp.float32),
            jax.ShapeDtypeStruct((N, FHALF), jnp.float32),
            jax.ShapeDtypeStruct((1, N), jnp.float32),
            jax.ShapeDtypeStruct((1, N), jnp.float32),
        ],
    )(outdeg_p, indeg_p, x, w1)


def _agg_body(src_hbm, dst_hbm, h1lo_hbm, h1hi_hbm, ndst_hbm, zeros_hbm,
              agg_hbm, coef_hbm, src_v, dst_v, rows_v, ndst_v, coef_v,
              shared_agg, sem):
    cid = lax.axis_index("c")
    sid = lax.axis_index("s")
    wid = cid * NS + sid

    pltpu.async_copy(src_hbm.at[sid], src_v, sem).wait()
    pltpu.async_copy(dst_hbm.at[sid], dst_v, sem).wait()
    pltpu.async_copy(ndst_hbm.at[0], ndst_v, sem).wait()

    zero16 = jnp.zeros((LANES,), jnp.float32)

    @pl.loop(0, N, step=LANES)
    def _(j):
        coef_v[pl.ds(j, LANES)] = zero16

    pltpu.async_copy(
        zeros_hbm.at[pl.ds(sid * ROWS_PT, ROWS_PT)],
        shared_agg.at[pl.ds(sid * ROWS_PT, ROWS_PT)],
        sem,
    ).wait()

    @pl.when(sid == NS - 1)
    def _():
        pltpu.async_copy(
            zeros_hbm.at[pl.ds(ROWS_PT * NS, ROWS_TAIL)],
            shared_agg.at[pl.ds(ROWS_PT * NS, ROWS_TAIL)],
            sem,
        ).wait()

    plsc.subcore_barrier()

    coef_lo = cid * (NCHUNK2 // NC)
    coef_hi = coef_lo + NCHUNK2 // NC

    @pl.loop(0, NCHUNK2)
    def _(c):
        @pl.when(cid == 0)
        def _():
            pltpu.async_copy(h1lo_hbm.at[src_v.at[c, 0]], rows_v, sem).wait()

        @pl.when(cid == 1)
        def _():
            pltpu.async_copy(h1hi_hbm.at[src_v.at[c, 0]], rows_v, sem).wait()

        pltpu.sync_copy(rows_v, shared_agg.at[dst_v.at[c, 0]], add=True)

        @pl.when(jnp.logical_and(c >= coef_lo, c < coef_hi))
        def _():
            @pl.loop(0, CHUNK, step=LANES)
            def _(i):
                d16 = dst_v[c, 0, pl.ds(i, LANES)]
                s16 = src_v[c, 0, pl.ds(i, LANES)]
                vals = plsc.load_gather(ndst_v, [d16])
                plsc.addupdate_scatter(coef_v, [s16], vals)

    plsc.subcore_barrier()
    pltpu.async_copy(
        shared_agg.at[pl.ds(sid * ROWS_PT, ROWS_PT)],
        agg_hbm.at[cid, pl.ds(sid * ROWS_PT, ROWS_PT)],
        sem,
    ).wait()

    @pl.when(sid == NS - 1)
    def _():
        pltpu.async_copy(
            shared_agg.at[pl.ds(ROWS_PT * NS, ROWS_TAIL)],
            agg_hbm.at[cid, pl.ds(ROWS_PT * NS, ROWS_TAIL)],
            sem,
        ).wait()

    pltpu.async_copy(coef_v, coef_hbm.at[wid, 0], sem).wait()


_agg_call = pl.kernel(
    _agg_body,
    out_type=[
        jax.ShapeDtypeStruct((NC, N, FHALF), jnp.float32),
        jax.ShapeDtypeStruct((NW, 1, N), jnp.float32),
    ],
    mesh=_mesh,
    scratch_types=[
        pltpu.VMEM((NCHUNK2, 1, CHUNK), jnp.int32),
        pltpu.VMEM((NCHUNK2, 1, CHUNK), jnp.int32),
        pltpu.VMEM((CHUNK, FHALF), jnp.float32),
        pltpu.VMEM((N,), jnp.float32),
        pltpu.VMEM((N,), jnp.float32),
        pltpu.VMEM_SHARED((N, FHALF), jnp.float32),
        pltpu.SemaphoreType.DMA,
    ],
    compiler_params=_sc_params_untiled,
)


def _final_body(agg_ref, coef_ref, nsrc_ref, ndst_ref, b1_ref, w2_ref, b2_ref,
                out_ref):
    agg = jnp.concatenate([agg_ref[0], agg_ref[1]], axis=1)
    h = jnp.maximum(agg * ndst_ref[0][:, None] + b1_ref[...], 0.0)
    coef = jnp.sum(coef_ref[...], axis=(0, 1))
    w = coef * nsrc_ref[0]
    ws = jnp.sum(h * w[:, None], axis=0, keepdims=True)
    total = jnp.sum(ws * w2_ref[...])
    out_ref[...] = total * (1.0 / N) + b2_ref[...]


def _final_call(aggp, coefp, nsrc, ndst, b1r, w2r, b2r):
    return pl.pallas_call(
        _final_body,
        out_shape=jax.ShapeDtypeStruct((1, 1), jnp.float32),
    )(aggp, coefp, nsrc, ndst, b1r, w2r, b2r)


@jax.jit
def _gcn(x, edge_index, W1, b1, W2, b2):
    ei = edge_index.astype(jnp.int32)
    src1 = ei[0].reshape(NW, NCHUNK1, 1, CHUNK)
    dst1 = ei[1].reshape(NW, NCHUNK1, 1, CHUNK)
    src2 = ei[0].reshape(NS, NCHUNK2, 1, CHUNK)
    dst2 = ei[1].reshape(NS, NCHUNK2, 1, CHUNK)
    outdeg_p, indeg_p = _deg_call(src1, dst1)
    h1lo, h1hi, nsrc, ndst = _mm1_call(outdeg_p, indeg_p, x, W1)
    zeros = jnp.zeros((N, FHALF), jnp.float32)
    aggp, coefp = _agg_call(src2, dst2, h1lo, h1hi, ndst, zeros)
    b1r = b1.reshape(1, FH)
    w2r = W2.reshape(1, FH)
    b2r = b2.reshape(1, 1)
    return _final_call(aggp, coefp, nsrc, ndst, b1r, w2r, b2r)


def kernel(x, edge_index, W1, b1, W2, b2):
    return _gcn(x, edge_index, W1, b1, W2, b2)
